# initial kernel scaffold (unmeasured)
import jax
import jax.numpy as jnp
from jax import lax
from jax.experimental import pallas as pl
from jax.experimental.pallas import tpu as pltpu

N_DEV = 16


def kernel(x, w_mat, scale_x, scale_w):
    m_total, k_shard = x.shape
    k_total, n = w_mat.shape
    m_per = m_total // N_DEV

    def body(x_ref, w_ref, sx_ref, sw_ref, out_ref,
             xs_ref, xt_ref, wc_ref, send_sems, recv_sems):
        me = lax.axis_index("i")

        xs_ref[...] = x_ref[...].astype(jnp.float8_e4m3fn)

        sends = []
        for d in range(1, N_DEV):
            p = lax.rem(me + d, N_DEV)
            rdma = pltpu.make_async_remote_copy(
                src_ref=xs_ref.at[pl.ds(p * m_per, m_per), :],
                dst_ref=xt_ref.at[:, pl.ds(me * k_shard, k_shard)],
                send_sem=send_sems.at[d],
                recv_sem=recv_sems.at[d],
                device_id=(p,),
                device_id_type=pl.DeviceIdType.MESH,
            )
            rdma.start()
            sends.append(rdma)

        xt_ref[:, pl.ds(me * k_shard, k_shard)] = xs_ref[pl.ds(me * m_per, m_per), :]
        wc_ref[...] = w_ref[...].astype(jnp.float8_e4m3fn)

        for d in range(1, N_DEV):
            s = lax.rem(me - d + N_DEV, N_DEV)
            recv = pltpu.make_async_remote_copy(
                src_ref=xs_ref.at[pl.ds(0, m_per), :],
                dst_ref=xt_ref.at[:, pl.ds(s * k_shard, k_shard)],
                send_sem=send_sems.at[d],
                recv_sem=recv_sems.at[d],
                device_id=(s,),
                device_id_type=pl.DeviceIdType.MESH,
            )
            recv.wait_recv()

        acc = jnp.dot(xt_ref[...], wc_ref[...],
                      preferred_element_type=jnp.float32)
        y = acc * (sx_ref[0] * sw_ref[0])
        z = jnp.clip(y, -60.0, 60.0)
        out_ref[...] = y / (1.0 + jnp.exp(-z))

        for rdma in sends:
            rdma.wait_send()

    return pl.pallas_call(
        body,
        out_shape=jax.ShapeDtypeStruct((m_per, n), jnp.float32),
        in_specs=[
            pl.BlockSpec(memory_space=pltpu.VMEM),
            pl.BlockSpec(memory_space=pltpu.VMEM),
            pl.BlockSpec(memory_space=pltpu.SMEM),
            pl.BlockSpec(memory_space=pltpu.SMEM),
        ],
        out_specs=pl.BlockSpec(memory_space=pltpu.VMEM),
        scratch_shapes=[
            pltpu.VMEM((m_total, k_shard), jnp.float8_e4m3fn),
            pltpu.VMEM((m_per, k_total), jnp.float8_e4m3fn),
            pltpu.VMEM((k_total, n), jnp.float8_e4m3fn),
            pltpu.SemaphoreType.DMA((N_DEV,)),
            pltpu.SemaphoreType.DMA((N_DEV,)),
        ],
        compiler_params=pltpu.CompilerParams(collective_id=0),
    )(x, w_mat, scale_x, scale_w)


# baseline (device time: 38952 ns/iter reference)
import jax
import jax.numpy as jnp
from jax import lax
from jax.experimental import pallas as pl
from jax.experimental.pallas import tpu as pltpu

N_DEV = 16


def kernel(x, w_mat, scale_x, scale_w):
    m_total, k_shard = x.shape
    k_total, n = w_mat.shape
    m_per = m_total // N_DEV

    def body(x_ref, w_ref, sx_ref, sw_ref, out_ref,
             xs_ref, xt_ref, wc_ref, send_sems, recv_sems):
        me = lax.axis_index("i")

        xs_ref[...] = x_ref[...].astype(jnp.float8_e4m3fn)

        sends = []
        for d in range(1, N_DEV):
            p = lax.rem(me + d, N_DEV)
            rdma = pltpu.make_async_remote_copy(
                src_ref=xs_ref.at[pl.ds(p * m_per, m_per), :],
                dst_ref=xt_ref.at[:, pl.ds(me * k_shard, k_shard)],
                send_sem=send_sems.at[d],
                recv_sem=recv_sems.at[d],
                device_id=(p,),
                device_id_type=pl.DeviceIdType.MESH,
            )
            rdma.start()
            sends.append(rdma)

        xt_ref[:, pl.ds(me * k_shard, k_shard)] = xs_ref[pl.ds(me * m_per, m_per), :]
        wc_ref[...] = w_ref[...].astype(jnp.float8_e4m3fn)

        for d in range(1, N_DEV):
            s = lax.rem(me - d + N_DEV, N_DEV)
            recv = pltpu.make_async_remote_copy(
                src_ref=xs_ref.at[pl.ds(0, m_per), :],
                dst_ref=xt_ref.at[:, pl.ds(s * k_shard, k_shard)],
                send_sem=send_sems.at[d],
                recv_sem=recv_sems.at[d],
                device_id=(s,),
                device_id_type=pl.DeviceIdType.MESH,
            )
            recv.wait_recv()

        acc = jnp.dot(xt_ref[...], wc_ref[...],
                      preferred_element_type=jnp.float32)
        y = acc * (sx_ref[0] * sw_ref[0])
        z = jnp.clip(y, -60.0, 60.0)
        out_ref[...] = y / (1.0 + jnp.exp(-z))

        for rdma in sends:
            rdma.wait_send()

    return pl.pallas_call(
        body,
        out_shape=jax.ShapeDtypeStruct((m_per, n), jnp.float32),
        in_specs=[
            pl.BlockSpec(memory_space=pltpu.VMEM),
            pl.BlockSpec(memory_space=pltpu.VMEM),
            pl.BlockSpec(memory_space=pltpu.SMEM),
            pl.BlockSpec(memory_space=pltpu.SMEM),
        ],
        out_specs=pl.BlockSpec(memory_space=pltpu.VMEM),
        scratch_shapes=[
            pltpu.VMEM((m_total, k_shard), jnp.float8_e4m3fn),
            pltpu.VMEM((m_per, k_total), jnp.float8_e4m3fn),
            pltpu.VMEM((k_total, n), jnp.float8_e4m3fn),
            pltpu.SemaphoreType.DMA((N_DEV,)),
            pltpu.SemaphoreType.DMA((N_DEV,)),
        ],
        compiler_params=pltpu.CompilerParams(
            vmem_limit_bytes=100 * 1024 * 1024,
        ),
    )(x, w_mat, scale_x, scale_w)


# device time: 27573 ns/iter; 1.4127x vs baseline; 1.4127x over previous
import jax
import jax.numpy as jnp
from jax import lax
from jax.experimental import pallas as pl
from jax.experimental.pallas import tpu as pltpu

N_DEV = 16
N_CHUNK = 8
SEND_ORDER = list(range(1, N_DEV))


def kernel(x, w_mat, scale_x, scale_w):
    m_total, k_shard = x.shape
    k_total, n = w_mat.shape
    m_per = m_total // N_DEV

    def body(x_ref, w_ref, sx_ref, sw_ref, out_ref,
             xs_ref, xt_ref, wc_ref, wtmp_ref,
             send_sems, recv_sems, dma_sems):
        me = lax.axis_index("i")

        xs_ref[...] = x_ref[...].astype(jnp.float8_e4m3fn)

        barrier_sem = pltpu.get_barrier_semaphore()
        for d in range(1, N_DEV):
            pl.semaphore_signal(barrier_sem, inc=1,
                                device_id=(lax.rem(me + d, N_DEV),),
                                device_id_type=pl.DeviceIdType.MESH)
        pl.semaphore_wait(barrier_sem, N_DEV - 1)

        sends = []
        for d in SEND_ORDER:
            p = lax.rem(me + d, N_DEV)
            rdma = pltpu.make_async_remote_copy(
                src_ref=xs_ref.at[pl.ds(p * m_per, m_per), :],
                dst_ref=xt_ref.at[:, pl.ds(me * k_shard, k_shard)],
                send_sem=send_sems.at[d],
                recv_sem=recv_sems.at[d],
                device_id=(p,),
                device_id_type=pl.DeviceIdType.MESH,
            )
            rdma.start()
            sends.append(rdma)

        xt_ref[:, pl.ds(me * k_shard, k_shard)] = xs_ref[pl.ds(me * m_per, m_per), :]

        rows = k_total // N_CHUNK

        def chunk_copy(c, slot):
            return pltpu.make_async_copy(
                w_ref.at[pl.ds(c * rows, rows), :],
                wtmp_ref.at[slot],
                dma_sems.at[slot],
            )

        chunk_copy(0, 0).start()
        for c in range(N_CHUNK):
            slot = c % 2
            if c + 1 < N_CHUNK:
                chunk_copy(c + 1, 1 - slot).start()
            chunk_copy(c, slot).wait()
            wc_ref[pl.ds(c * rows, rows), :] = \
                wtmp_ref[slot].astype(jnp.float8_e4m3fn)

        for d in range(1, N_DEV):
            s = lax.rem(me - d + N_DEV, N_DEV)
            recv = pltpu.make_async_remote_copy(
                src_ref=xs_ref.at[pl.ds(0, m_per), :],
                dst_ref=xt_ref.at[:, pl.ds(s * k_shard, k_shard)],
                send_sem=send_sems.at[d],
                recv_sem=recv_sems.at[d],
                device_id=(s,),
                device_id_type=pl.DeviceIdType.MESH,
            )
            recv.wait_recv()

        acc = jnp.dot(xt_ref[...], wc_ref[...],
                      preferred_element_type=jnp.float32)
        y = acc * (sx_ref[0] * sw_ref[0])
        z = jnp.clip(y, -60.0, 60.0)
        out_ref[...] = y / (1.0 + jnp.exp(-z))

        for rdma in sends:
            rdma.wait_send()

    return pl.pallas_call(
        body,
        out_shape=jax.ShapeDtypeStruct((m_per, n), jnp.float32),
        in_specs=[
            pl.BlockSpec(memory_space=pltpu.VMEM),
            pl.BlockSpec(memory_space=pl.ANY),
            pl.BlockSpec(memory_space=pltpu.SMEM),
            pl.BlockSpec(memory_space=pltpu.SMEM),
        ],
        out_specs=pl.BlockSpec(memory_space=pltpu.VMEM),
        scratch_shapes=[
            pltpu.VMEM((m_total, k_shard), jnp.float8_e4m3fn),
            pltpu.VMEM((m_per, k_total), jnp.float8_e4m3fn),
            pltpu.VMEM((k_total, n), jnp.float8_e4m3fn),
            pltpu.VMEM((2, k_total // N_CHUNK, n), jnp.float32),
            pltpu.SemaphoreType.DMA((N_DEV,)),
            pltpu.SemaphoreType.DMA((N_DEV,)),
            pltpu.SemaphoreType.DMA((2,)),
        ],
        compiler_params=pltpu.CompilerParams(
            vmem_limit_bytes=100 * 1024 * 1024,
            collective_id=0,
        ),
    )(x, w_mat, scale_x, scale_w)


# device time: 24977 ns/iter; 1.5595x vs baseline; 1.1039x over previous
import jax
import jax.numpy as jnp
from jax import lax
from jax.experimental import pallas as pl
from jax.experimental.pallas import tpu as pltpu

N_DEV = 16
N_SLOT = 4
QSEQ = [0] + list(range(8, 16)) + list(range(1, 8))


def kernel(x, w_mat, scale_x, scale_w):
    m_total, k_shard = x.shape
    k_total, n = w_mat.shape
    m_per = m_total // N_DEV
    half = (N_DEV // 2) * k_shard

    def body(x_ref, w_ref, sx_ref, sw_ref, out_ref,
             xs_ref, xt_ref, wc_ref, wtmp_ref,
             send_sems, recv_sems, dma_sems):
        me = lax.axis_index("i")

        xs_ref[...] = x_ref[...].astype(jnp.float8_e4m3fn)

        barrier_sem = pltpu.get_barrier_semaphore()
        for d in range(1, N_DEV):
            pl.semaphore_signal(barrier_sem, inc=1,
                                device_id=(lax.rem(me + d, N_DEV),),
                                device_id_type=pl.DeviceIdType.MESH)
        pl.semaphore_wait(barrier_sem, N_DEV - 1)

        sends = []
        for d in range(1, N_DEV):
            p = lax.rem(me + d, N_DEV)
            rdma = pltpu.make_async_remote_copy(
                src_ref=xs_ref.at[pl.ds(p * m_per, m_per), :],
                dst_ref=xt_ref.at[:, pl.ds((N_DEV - d) * k_shard, k_shard)],
                send_sem=send_sems.at[d],
                recv_sem=recv_sems.at[d],
                device_id=(p,),
                device_id_type=pl.DeviceIdType.MESH,
            )
            rdma.start()
            sends.append(rdma)

        xt_ref[:, pl.ds(0, k_shard)] = xs_ref[pl.ds(me * m_per, m_per), :]

        def w_dma(i, slot):
            q = QSEQ[i]
            s = lax.rem(me + q, N_DEV)
            return pltpu.make_async_copy(
                w_ref.at[pl.ds(s * k_shard, k_shard), :],
                wtmp_ref.at[slot],
                dma_sems.at[slot],
            )

        for i in range(N_SLOT):
            w_dma(i, i).start()

        def cast_step(i):
            slot = i % N_SLOT
            w_dma(i, slot).wait()
            q = QSEQ[i]
            wc_ref[pl.ds(q * k_shard, k_shard), :] = \
                wtmp_ref[slot].astype(jnp.float8_e4m3fn)
            if i + N_SLOT < N_DEV:
                w_dma(i + N_SLOT, slot).start()

        def wait_recv(d):
            recv = pltpu.make_async_remote_copy(
                src_ref=xs_ref.at[pl.ds(0, m_per), :],
                dst_ref=xt_ref.at[:, pl.ds((N_DEV - d) * k_shard, k_shard)],
                send_sem=send_sems.at[d],
                recv_sem=recv_sems.at[d],
                device_id=(me,),
                device_id_type=pl.DeviceIdType.MESH,
            )
            recv.wait_recv()

        for i in range(9):
            cast_step(i)
        for d in range(1, 9):
            wait_recv(d)
        acc = jnp.dot(xt_ref[:, pl.ds(half, half)],
                      wc_ref[pl.ds(half, half), :],
                      preferred_element_type=jnp.float32)
        acc = acc + jnp.dot(xt_ref[:, pl.ds(0, k_shard)],
                            wc_ref[pl.ds(0, k_shard), :],
                            preferred_element_type=jnp.float32)
        out_ref[...] = acc

        for i in range(9, N_DEV):
            cast_step(i)
        for d in range(9, N_DEV):
            wait_recv(d)
        acc = out_ref[...] + jnp.dot(
            xt_ref[:, pl.ds(k_shard, half - k_shard)],
            wc_ref[pl.ds(k_shard, half - k_shard), :],
            preferred_element_type=jnp.float32)

        y = acc * (sx_ref[0] * sw_ref[0])
        z = jnp.clip(y, -60.0, 60.0)
        out_ref[...] = y / (1.0 + jnp.exp(-z))

        for rdma in sends:
            rdma.wait_send()

    return pl.pallas_call(
        body,
        out_shape=jax.ShapeDtypeStruct((m_per, n), jnp.float32),
        in_specs=[
            pl.BlockSpec(memory_space=pltpu.VMEM),
            pl.BlockSpec(memory_space=pl.ANY),
            pl.BlockSpec(memory_space=pltpu.SMEM),
            pl.BlockSpec(memory_space=pltpu.SMEM),
        ],
        out_specs=pl.BlockSpec(memory_space=pltpu.VMEM),
        scratch_shapes=[
            pltpu.VMEM((m_total, k_shard), jnp.float8_e4m3fn),
            pltpu.VMEM((m_per, k_total), jnp.float8_e4m3fn),
            pltpu.VMEM((k_total, n), jnp.float8_e4m3fn),
            pltpu.VMEM((N_SLOT, k_shard, n), jnp.float32),
            pltpu.SemaphoreType.DMA((N_DEV,)),
            pltpu.SemaphoreType.DMA((N_DEV,)),
            pltpu.SemaphoreType.DMA((N_SLOT,)),
        ],
        compiler_params=pltpu.CompilerParams(
            vmem_limit_bytes=100 * 1024 * 1024,
            collective_id=0,
        ),
    )(x, w_mat, scale_x, scale_w)
